# baseline (speedup 1.0000x reference)
import jax
import jax.numpy as jnp
from jax.experimental import pallas as pl

_N_POINTS = 8192
_N_TRIS = 2048
_B = 256


def _row(m, k):
    return m[k : k + 1, :]


def _block_kernel(ptsb_ref, t5_ref, scal_ref, pp_ref, d_ref, i_ref, t_ref):
    pts_b = ptsb_ref[...]
    t5 = t5_ref[...]
    scal = scal_ref[...]
    pp = pp_ref[...]

    a_ab = _row(scal, 0)
    a_ac = _row(scal, 1)
    b_ab = _row(scal, 2)
    b_ac = _row(scal, 3)
    c_ab = _row(scal, 4)
    c_ac = _row(scal, 5)
    aa = _row(scal, 6)
    cc_ = _row(scal, 7)
    bb = _row(scal, 8)
    ee = _row(scal, 9)
    a_a = _row(scal, 10)
    b_b = _row(scal, 11)
    c_c = _row(scal, 12)

    big = jnp.dot(pts_b, t5, preferred_element_type=jnp.float32)
    T = _N_TRIS
    p_ab = big[:, 0 * T : 1 * T]
    p_ac = big[:, 1 * T : 2 * T]
    p_m2a = big[:, 2 * T : 3 * T]
    p_m2b = big[:, 3 * T : 4 * T]
    p_m2c = big[:, 4 * T : 5 * T]
    d1 = p_ab - a_ab
    d2 = p_ac - a_ac
    d3 = p_ab - b_ab
    d4 = p_ac - b_ac
    d5 = p_ab - c_ab
    d6 = p_ac - c_ac
    ap2 = pp + p_m2a + a_a
    bp2 = pp + p_m2b + b_b
    cp2 = pp + p_m2c + c_c
    vc = d1 * d4 - d3 * d2
    vb = d5 * d2 - d1 * d6
    va = d3 * d6 - d5 * d4
    cond_a = (d1 <= 0) & (d2 <= 0)
    cond_b = (d3 >= 0) & (d4 <= d3)
    cond_c = (d6 >= 0) & (d5 <= d6)
    s1 = d4 - d3
    s2 = d5 - d6
    cond_ab = (vc <= 0) & (d1 >= 0) & (d3 <= 0)
    cond_ac = (vb <= 0) & (d2 >= 0) & (d6 <= 0)
    cond_bc = (va <= 0) & (s1 >= 0) & (s2 >= 0)
    num = jnp.where(cond_ab, d1, jnp.where(cond_ac, d2, s1))
    den_raw = jnp.where(cond_ab, d1 - d3, jnp.where(cond_ac, d2 - d6, s1 + s2))
    base = jnp.where(cond_ab | cond_ac, ap2, bp2)
    len_ = jnp.where(cond_ab, aa, jnp.where(cond_ac, cc_, ee))
    dens = jnp.where(den_raw != 0, den_raw, 1.0)
    t_e = num / dens
    dist_e = base - 2.0 * t_e * num + t_e * t_e * len_
    det = va + vb + vc
    dets = jnp.where(det != 0, det, 1.0)
    v = vb / dets
    w = vc / dets
    dist_f = (
        ap2 - 2.0 * v * d1 - 2.0 * w * d2 + v * v * aa + 2.0 * v * w * bb + w * w * cc_
    )
    edge = cond_ab | cond_ac | cond_bc
    dist = jnp.where(edge, dist_e, dist_f)
    dist = jnp.where(cond_c, cp2, dist)
    dist = jnp.where(cond_b, bp2, dist)
    dist = jnp.where(cond_a, ap2, dist)
    ttype = jnp.full(dist.shape, 6, dtype=jnp.int32)
    ttype = jnp.where(cond_bc, 5, ttype)
    ttype = jnp.where(cond_ac, 4, ttype)
    ttype = jnp.where(cond_ab, 3, ttype)
    ttype = jnp.where(cond_c, 2, ttype)
    ttype = jnp.where(cond_b, 1, ttype)
    ttype = jnp.where(cond_a, 0, ttype)

    dmin = jnp.maximum(jnp.min(dist, axis=1), 0.0)
    j = jax.lax.broadcasted_iota(jnp.int32, dist.shape, 1)
    code = j * 8 + ttype
    big_i = jnp.int32(1 << 30)
    cmin = jnp.min(jnp.where(dist <= dmin[:, None], code, big_i), axis=1)
    d_ref[...] = dmin
    i_ref[...] = cmin // 8
    t_ref[...] = cmin & 7


def kernel(points, verts_1, verts_2, verts_3):
    ab = verts_2 - verts_1
    ac = verts_3 - verts_1
    bc = verts_3 - verts_2
    scal = jnp.stack(
        [
            jnp.sum(verts_1 * ab, -1),
            jnp.sum(verts_1 * ac, -1),
            jnp.sum(verts_2 * ab, -1),
            jnp.sum(verts_2 * ac, -1),
            jnp.sum(verts_3 * ab, -1),
            jnp.sum(verts_3 * ac, -1),
            jnp.sum(ab * ab, -1),
            jnp.sum(ac * ac, -1),
            jnp.sum(ab * ac, -1),
            jnp.sum(bc * bc, -1),
            jnp.sum(verts_1 * verts_1, -1),
            jnp.sum(verts_2 * verts_2, -1),
            jnp.sum(verts_3 * verts_3, -1),
        ],
        axis=0,
    )
    pp = jnp.sum(points * points, -1)[:, None]
    t5f = jnp.concatenate(
        [ab.T, ac.T, -2.0 * verts_1.T, -2.0 * verts_2.T, -2.0 * verts_3.T], axis=1
    )
    pts_b = points.astype(jnp.bfloat16)
    return _run(pts_b, t5f.astype(jnp.bfloat16), scal, pp)


def _run(pts_b, t5, scal, pp):
    n = pts_b.shape[0]
    nb = n // _B
    d, i, t = pl.pallas_call(
        _block_kernel,
        grid=(nb,),
        in_specs=[
            pl.BlockSpec((_B, 3), lambda k: (k, 0)),
            pl.BlockSpec((3, 5 * _N_TRIS), lambda k: (0, 0)),
            pl.BlockSpec((13, _N_TRIS), lambda k: (0, 0)),
            pl.BlockSpec((_B, 1), lambda k: (k, 0)),
        ],
        out_specs=[
            pl.BlockSpec((_B,), lambda k: (k,)),
            pl.BlockSpec((_B,), lambda k: (k,)),
            pl.BlockSpec((_B,), lambda k: (k,)),
        ],
        out_shape=[
            jax.ShapeDtypeStruct((n,), jnp.float32),
            jax.ShapeDtypeStruct((n,), jnp.int32),
            jax.ShapeDtypeStruct((n,), jnp.int32),
        ],
    )(pts_b, t5, scal, pp)
    return d, i, t


# speedup vs baseline: 1.0050x; 1.0050x over previous
import jax
import jax.numpy as jnp
from jax.experimental import pallas as pl

_N_POINTS = 8192
_N_TRIS = 2048
_B = 256


def _row(m, k):
    return m[k : k + 1, :]


def _block_kernel(ptsb_ref, t5_ref, scal_ref, pp_ref, d_ref, i_ref, t_ref):
    pts_b = ptsb_ref[...]
    t5 = t5_ref[...]
    scal = scal_ref[...]
    pp = pp_ref[...]

    a_ab = _row(scal, 0)
    a_ac = _row(scal, 1)
    b_ab = _row(scal, 2)
    b_ac = _row(scal, 3)
    c_ab = _row(scal, 4)
    c_ac = _row(scal, 5)
    aa = _row(scal, 6)
    cc_ = _row(scal, 7)
    bb = _row(scal, 8)
    ee = _row(scal, 9)
    a_a = _row(scal, 10)
    b_b = _row(scal, 11)
    c_c = _row(scal, 12)

    big = jnp.dot(pts_b, t5, preferred_element_type=jnp.float32)
    T = _N_TRIS
    p_ab = big[:, 0 * T : 1 * T]
    p_ac = big[:, 1 * T : 2 * T]
    p_m2a = big[:, 2 * T : 3 * T]
    p_m2b = big[:, 3 * T : 4 * T]
    p_m2c = big[:, 4 * T : 5 * T]
    d1 = p_ab - a_ab
    d2 = p_ac - a_ac
    d3 = p_ab - b_ab
    d4 = p_ac - b_ac
    d5 = p_ab - c_ab
    d6 = p_ac - c_ac
    ap2 = pp + p_m2a + a_a
    bp2 = pp + p_m2b + b_b
    cp2 = pp + p_m2c + c_c
    vc = d1 * d4 - d3 * d2
    vb = d5 * d2 - d1 * d6
    va = d3 * d6 - d5 * d4
    cond_a = (d1 <= 0) & (d2 <= 0)
    cond_b = (d3 >= 0) & (d4 <= d3)
    cond_c = (d6 >= 0) & (d5 <= d6)
    s1 = d4 - d3
    s2 = d5 - d6
    cond_ab = (vc <= 0) & (d1 >= 0) & (d3 <= 0)
    cond_ac = (vb <= 0) & (d2 >= 0) & (d6 <= 0)
    cond_bc = (va <= 0) & (s1 >= 0) & (s2 >= 0)
    num = jnp.where(cond_ab, d1, jnp.where(cond_ac, d2, s1))
    den_raw = jnp.where(cond_ab, d1 - d3, jnp.where(cond_ac, d2 - d6, s1 + s2))
    base = jnp.where(cond_ab | cond_ac, ap2, bp2)
    len_ = jnp.where(cond_ab, aa, jnp.where(cond_ac, cc_, ee))
    dens = jnp.where(den_raw != 0, den_raw, 1.0)
    t_e = num / dens
    dist_e = base - 2.0 * t_e * num + t_e * t_e * len_
    det = va + vb + vc
    dets = jnp.where(det != 0, det, 1.0)
    v = vb / dets
    w = vc / dets
    dist_f = (
        ap2 - 2.0 * v * d1 - 2.0 * w * d2 + v * v * aa + 2.0 * v * w * bb + w * w * cc_
    )
    edge = cond_ab | cond_ac | cond_bc
    dist = jnp.where(edge, dist_e, dist_f)
    dist = jnp.where(cond_c, cp2, dist)
    dist = jnp.where(cond_b, bp2, dist)
    dist = jnp.where(cond_a, ap2, dist)
    ttype = jnp.where(cond_bc, jnp.int32(5), jnp.int32(6))
    ttype = jnp.where(cond_ac, 4, ttype)
    ttype = jnp.where(cond_ab, 3, ttype)
    ttype = jnp.where(cond_c, 2, ttype)
    ttype = jnp.where(cond_b, 1, ttype)
    ttype = jnp.where(cond_a, 0, ttype)

    dmin = jnp.maximum(jnp.min(dist, axis=1), 0.0)
    j = jax.lax.broadcasted_iota(jnp.int32, dist.shape, 1)
    code = j * 8 + ttype
    big_i = jnp.int32(1 << 30)
    cmin = jnp.min(jnp.where(dist <= dmin[:, None], code, big_i), axis=1)
    d_ref[...] = dmin
    i_ref[...] = cmin // 8
    t_ref[...] = cmin & 7


def kernel(points, verts_1, verts_2, verts_3):
    ab = verts_2 - verts_1
    ac = verts_3 - verts_1
    bc = verts_3 - verts_2
    scal = jnp.stack(
        [
            jnp.sum(verts_1 * ab, -1),
            jnp.sum(verts_1 * ac, -1),
            jnp.sum(verts_2 * ab, -1),
            jnp.sum(verts_2 * ac, -1),
            jnp.sum(verts_3 * ab, -1),
            jnp.sum(verts_3 * ac, -1),
            jnp.sum(ab * ab, -1),
            jnp.sum(ac * ac, -1),
            jnp.sum(ab * ac, -1),
            jnp.sum(bc * bc, -1),
            jnp.sum(verts_1 * verts_1, -1),
            jnp.sum(verts_2 * verts_2, -1),
            jnp.sum(verts_3 * verts_3, -1),
        ],
        axis=0,
    )
    pp = jnp.sum(points * points, -1)[:, None]
    t5f = jnp.concatenate(
        [ab.T, ac.T, -2.0 * verts_1.T, -2.0 * verts_2.T, -2.0 * verts_3.T], axis=1
    )
    pts_b = points.astype(jnp.bfloat16)
    return _run(pts_b, t5f.astype(jnp.bfloat16), scal, pp)


def _run(pts_b, t5, scal, pp):
    n = pts_b.shape[0]
    nb = n // _B
    d, i, t = pl.pallas_call(
        _block_kernel,
        grid=(nb,),
        in_specs=[
            pl.BlockSpec((_B, 3), lambda k: (k, 0)),
            pl.BlockSpec((3, 5 * _N_TRIS), lambda k: (0, 0)),
            pl.BlockSpec((13, _N_TRIS), lambda k: (0, 0)),
            pl.BlockSpec((_B, 1), lambda k: (k, 0)),
        ],
        out_specs=[
            pl.BlockSpec((_B,), lambda k: (k,)),
            pl.BlockSpec((_B,), lambda k: (k,)),
            pl.BlockSpec((_B,), lambda k: (k,)),
        ],
        out_shape=[
            jax.ShapeDtypeStruct((n,), jnp.float32),
            jax.ShapeDtypeStruct((n,), jnp.int32),
            jax.ShapeDtypeStruct((n,), jnp.int32),
        ],
    )(pts_b, t5, scal, pp)
    return d, i, t
